# Initial kernel scaffold; baseline (speedup 1.0000x reference)
import jax
import jax.numpy as jnp
from jax.experimental import pallas as pl


def kernel(cost):
    raise NotImplementedError("write your pallas kernel here")


# speedup vs baseline: 15.1873x; 15.1873x over previous
import functools
import jax
import jax.numpy as jnp
from jax import lax
from jax.experimental import pallas as pl
from jax.experimental.pallas import tpu as pltpu
from jax.experimental.pallas import tpu_sc as plsc

_B, _D, _H, _W = 8, 48, 160, 320
_HW = _H * _W
_PIX = _B * _HW
_NW = 32
_PPW = _PIX // _NW
_C = 640
_NCHUNK = _PPW // _C
_LANES = 16


def _body(cost_hbm, out_hbm, buf, obuf, sem):
    nc = 2
    wid = lax.axis_index("s") * nc + lax.axis_index("c")
    b = wid // 4
    hw_base = (wid % 4) * _PPW
    row0 = b * _D

    def chunk(c, _):
        col = hw_base + c * _C
        pltpu.sync_copy(cost_hbm.at[pl.ds(row0, _D), pl.ds(col, _C)], buf)

        def vec(j, _):
            j16 = j * _LANES
            v = buf[0, pl.ds(j16, _LANES)]
            max1 = v
            i1 = jnp.zeros((_LANES,), jnp.float32)
            max2 = jnp.full((_LANES,), -jnp.inf, jnp.float32)
            i2 = jnp.zeros((_LANES,), jnp.float32)
            for d in range(1, _D):
                v = buf[d, pl.ds(j16, _LANES)]
                df = jnp.float32(d)
                gt1 = v > max1
                gt2 = v > max2
                i2 = jnp.where(gt1, i1, jnp.where(gt2, df, i2))
                max2 = jnp.where(gt1, max1, jnp.where(gt2, v, max2))
                i1 = jnp.where(gt1, df, i1)
                max1 = jnp.where(gt1, v, max1)
            e = jnp.exp(max2 - max1)
            obuf[pl.ds(j16, _LANES)] = (i1 + i2 * e) / (1.0 + e)
            return 0

        lax.fori_loop(0, _C // _LANES, vec, 0)
        pltpu.sync_copy(obuf, out_hbm.at[pl.ds(wid * _PPW + c * _C, _C)])
        return 0

    lax.fori_loop(0, _NCHUNK, chunk, 0)


@jax.jit
def kernel(cost):
    cost2d = cost.reshape(_B * _D, _HW)
    mesh = plsc.VectorSubcoreMesh(
        core_axis_name="c", subcore_axis_name="s", num_cores=2, num_subcores=16
    )
    out = pl.kernel(
        _body,
        out_type=jax.ShapeDtypeStruct((_PIX,), jnp.float32),
        mesh=mesh,
        scratch_types=[
            pltpu.VMEM((_D, _C), jnp.float32),
            pltpu.VMEM((_C,), jnp.float32),
            pltpu.SemaphoreType.DMA,
        ],
    )(cost2d)
    return out.reshape(_B, 1, _H, _W)


# speedup vs baseline: 17.6376x; 1.1613x over previous
import functools
import jax
import jax.numpy as jnp
from jax import lax
from jax.experimental import pallas as pl
from jax.experimental.pallas import tpu as pltpu
from jax.experimental.pallas import tpu_sc as plsc

_B, _D, _H, _W = 8, 48, 160, 320
_HW = _H * _W
_PIX = _B * _HW
_NW = 32
_PPW = _PIX // _NW
_C = 640
_NCHUNK = _PPW // _C
_LANES = 16


def _body(cost_hbm, out_hbm, buf, obuf, isem0, isem1, osem0, osem1):
    nc = 2
    wid = lax.axis_index("s") * nc + lax.axis_index("c")
    b = wid // 4
    hw_base = (wid % 4) * _PPW
    row0 = b * _D
    isems = (isem0, isem1)
    osems = (osem0, osem1)

    def in_copy(c, slot):
        src = cost_hbm.at[pl.ds(row0, _D), pl.ds(hw_base + c * _C, _C)]
        return pltpu.make_async_copy(src, buf.at[slot], isems[slot])

    def out_copy(c, slot):
        dst = out_hbm.at[pl.ds(wid * _PPW + c * _C, _C)]
        return pltpu.make_async_copy(obuf.at[slot], dst, osems[slot])

    def compute(slot):
        def vec(j, _):
            j16 = j * _LANES
            v = buf[slot, 0, pl.ds(j16, _LANES)]
            max1 = v
            i1 = jnp.zeros((_LANES,), jnp.float32)
            max2 = jnp.full((_LANES,), -jnp.inf, jnp.float32)
            i2 = jnp.zeros((_LANES,), jnp.float32)
            for d in range(1, _D):
                v = buf[slot, d, pl.ds(j16, _LANES)]
                df = jnp.float32(d)
                gt1 = v > max1
                gt2 = v > max2
                i2 = jnp.where(gt1, i1, jnp.where(gt2, df, i2))
                max2 = jnp.where(gt1, max1, jnp.where(gt2, v, max2))
                i1 = jnp.where(gt1, df, i1)
                max1 = jnp.where(gt1, v, max1)
            e = jnp.exp(max2 - max1)
            obuf[slot, pl.ds(j16, _LANES)] = (i1 + i2 * e) / (1.0 + e)
            return 0

        lax.fori_loop(0, _C // _LANES, vec, 0, unroll=2)

    npairs = _NCHUNK // 2
    in_copy(0, 0).start()

    def pair(g, _):
        c0 = 2 * g
        in_copy(c0, 0).wait()
        in_copy(c0 + 1, 1).start()

        @pl.when(g > 0)
        def _():
            out_copy(c0 - 2, 0).wait()

        compute(0)
        out_copy(c0, 0).start()

        in_copy(c0 + 1, 1).wait()

        @pl.when(g + 1 < npairs)
        def _():
            in_copy(c0 + 2, 0).start()

        @pl.when(g > 0)
        def _():
            out_copy(c0 - 1, 1).wait()

        compute(1)
        out_copy(c0 + 1, 1).start()
        return 0

    lax.fori_loop(0, npairs, pair, 0)
    out_copy(_NCHUNK - 2, 0).wait()
    out_copy(_NCHUNK - 1, 1).wait()


@jax.jit
def kernel(cost):
    cost2d = cost.reshape(_B * _D, _HW)
    mesh = plsc.VectorSubcoreMesh(
        core_axis_name="c", subcore_axis_name="s", num_cores=2, num_subcores=16
    )
    out = pl.kernel(
        _body,
        out_type=jax.ShapeDtypeStruct((_PIX,), jnp.float32),
        mesh=mesh,
        scratch_types=[
            pltpu.VMEM((2, _D, _C), jnp.float32),
            pltpu.VMEM((2, _C), jnp.float32),
            pltpu.SemaphoreType.DMA,
            pltpu.SemaphoreType.DMA,
            pltpu.SemaphoreType.DMA,
            pltpu.SemaphoreType.DMA,
        ],
    )(cost2d)
    return out.reshape(_B, 1, _H, _W)


# speedup vs baseline: 27.6902x; 1.5700x over previous
import jax
import jax.numpy as jnp
from jax import lax
from jax.experimental import pallas as pl
from jax.experimental.pallas import tpu as pltpu
from jax.experimental.pallas import tpu_sc as plsc

_B, _D, _H, _W = 8, 48, 160, 320
_ROWS = 8
_CPX = _ROWS * _W
_NCHUNK = 5
_DQ = 12
_NQ = _D // _DQ
_LANES = 16
_GROUPS = _CPX // _LANES
_WPB = 4


def _body(cost_hbm, out_hbm, buf, st, obuf, isem0, isem1, osem):
    nc = 2
    wid = lax.axis_index("s") * nc + lax.axis_index("c")
    b = wid // _WPB
    row0 = (wid % _WPB) * (_NCHUNK * _ROWS)
    isems = (isem0, isem1)

    def in_copy(c, q, slot):
        src = cost_hbm.at[b, pl.ds(q * _DQ, _DQ), pl.ds(row0 + c * _ROWS, _ROWS), :]
        return pltpu.make_async_copy(src, buf.at[slot], isems[slot])

    def out_copy(c):
        dst = out_hbm.at[b, 0, pl.ds(row0 + c * _ROWS, _ROWS), :]
        return pltpu.make_async_copy(obuf.at[pl.ds(c * _ROWS, _ROWS), :], dst, osem)

    def compute_tile(q, slot, c):
        def body(j, _):
            hh = j // (_W // _LANES)
            w16 = (j % (_W // _LANES)) * _LANES
            s16 = j * _LANES
            if q == 0:
                v = buf[slot, 0, hh, pl.ds(w16, _LANES)]
                max1 = v
                i1 = jnp.zeros((_LANES,), jnp.float32)
                max2 = jnp.full((_LANES,), -jnp.inf, jnp.float32)
                i2 = jnp.zeros((_LANES,), jnp.float32)
                dds = range(1, _DQ)
            else:
                max1 = st[0, pl.ds(s16, _LANES)]
                max2 = st[1, pl.ds(s16, _LANES)]
                i1 = st[2, pl.ds(s16, _LANES)]
                i2 = st[3, pl.ds(s16, _LANES)]
                dds = range(_DQ)
            for dd in dds:
                v = buf[slot, dd, hh, pl.ds(w16, _LANES)]
                df = jnp.float32(q * _DQ + dd)
                gt1 = v > max1
                gt2 = v > max2
                i2 = jnp.where(gt1, i1, jnp.where(gt2, df, i2))
                max2 = jnp.where(gt1, max1, jnp.where(gt2, v, max2))
                i1 = jnp.where(gt1, df, i1)
                max1 = jnp.where(gt1, v, max1)
            if q == _NQ - 1:
                e = jnp.exp(max2 - max1)
                obuf[c * _ROWS + hh, pl.ds(w16, _LANES)] = (i1 + i2 * e) / (1.0 + e)
            else:
                st[0, pl.ds(s16, _LANES)] = max1
                st[1, pl.ds(s16, _LANES)] = max2
                st[2, pl.ds(s16, _LANES)] = i1
                st[3, pl.ds(s16, _LANES)] = i2
            return 0

        lax.fori_loop(0, _GROUPS, body, 0, unroll=2)

    in_copy(0, 0, 0).start()

    def chunk(c, _):
        for q in range(_NQ):
            slot = q % 2
            in_copy(c, q, slot).wait()
            if q + 1 < _NQ:
                in_copy(c, q + 1, 1 - slot).start()
            else:

                @pl.when(c + 1 < _NCHUNK)
                def _():
                    in_copy(c + 1, 0, 1 - slot).start()

            compute_tile(q, slot, c)
        out_copy(c).start()
        return 0

    lax.fori_loop(0, _NCHUNK, chunk, 0)
    for _ in range(_NCHUNK):
        out_copy(0).wait()


@jax.jit
def kernel(cost):
    mesh = plsc.VectorSubcoreMesh(
        core_axis_name="c", subcore_axis_name="s", num_cores=2, num_subcores=16
    )
    return pl.kernel(
        _body,
        out_type=jax.ShapeDtypeStruct((_B, 1, _H, _W), jnp.float32),
        mesh=mesh,
        scratch_types=[
            pltpu.VMEM((2, _DQ, _ROWS, _W), jnp.float32),
            pltpu.VMEM((4, _CPX), jnp.float32),
            pltpu.VMEM((_NCHUNK * _ROWS, _W), jnp.float32),
            pltpu.SemaphoreType.DMA,
            pltpu.SemaphoreType.DMA,
            pltpu.SemaphoreType.DMA,
        ],
    )(cost)
